# baseline (speedup 1.0000x reference)
import dataclasses

import jax
import jax.numpy as jnp
from jax import lax
from jax.experimental import pallas as pl
from jax.experimental.pallas import tpu as pltpu
from jax.experimental.pallas import tpu_sc as plsc

BATCH = 16384
RANK = 32
NC = 2
NS = 16
LANES = 16
NW = NC * NS
BPW = BATCH // NW
QG = 1
SETS = 8
LAG = 7
SPW = LANES // QG
NT = BPW // LANES
BLK = 128


def _dot_gather_body(uidx_hbm, vidx_hbm, ut_hbm, vt_hbm, out_hbm,
                     idx_u, idx_v, cols_u, cols_v, blku, blkv, out_v,
                     sem_u, sem_v, sem_o):
    wid = lax.axis_index("s") * NC + lax.axis_index("c")
    base = wid * BPW

    cpu = pltpu.async_copy(uidx_hbm.at[pl.ds(base, BPW)], idx_u, sem_o)
    cpv = pltpu.async_copy(vidx_hbm.at[pl.ds(base, BPW)], idx_v, sem_o)
    cpu.wait()
    cpv.wait()

    iota = lax.iota(jnp.int32, LANES)

    def fire(t_hbm, blk, sem, i16, lane_base, p):
        for k in range(QG):
            c = pl.multiple_of((i16[lane_base + k] >> 7) * BLK, BLK)
            pltpu.async_copy(
                t_hbm.at[:, pl.ds(c, BLK)], blk.at[p, k], sem.at[p])

    def extract(t_hbm, blk, sem, cols, i16, lane_base, p, sbase):
        for k in range(QG):
            pltpu.make_async_copy(
                t_hbm.at[:, pl.ds(0, BLK)], blk.at[p, k], sem.at[p]).wait()
        pfull = jnp.full((LANES,), p, jnp.int32)
        for k in range(QG):
            b = sbase + k
            lane = jnp.full((LANES,), i16[lane_base + k] & (BLK - 1),
                            jnp.int32)
            kfull = jnp.full((LANES,), k, jnp.int32)
            lo = plsc.load_gather(blk, [pfull, kfull, iota, lane])
            hi = plsc.load_gather(blk, [pfull, kfull, LANES + iota, lane])
            plsc.store_scatter(cols, [iota * BPW + b], lo)
            plsc.store_scatter(cols, [(LANES + iota) * BPW + b], hi)

    @pl.loop(0, NT)
    def _(t):
        iu = idx_u[pl.ds(t * LANES, LANES)]
        iv = idx_v[pl.ds(t * LANES, LANES)]
        for q in range(SPW):
            p = q & (SETS - 1)
            fire(ut_hbm, blku, sem_u, iu, q * QG, p)
            fire(vt_hbm, blkv, sem_v, iv, q * QG, p)
            qq = q - LAG
            px = qq & (SETS - 1)
            sprev = t * LANES + qq * QG
            if qq < 0:
                @pl.when(t > 0)
                def _():
                    iup = idx_u[pl.ds((t - 1) * LANES, LANES)]
                    ivp = idx_v[pl.ds((t - 1) * LANES, LANES)]
                    extract(ut_hbm, blku, sem_u, cols_u, iup,
                            (qq + SPW) * QG, px, sprev)
                    extract(vt_hbm, blkv, sem_v, cols_v, ivp,
                            (qq + SPW) * QG, px, sprev)
            else:
                extract(ut_hbm, blku, sem_u, cols_u, iu,
                        qq * QG, px, sprev)
                extract(vt_hbm, blkv, sem_v, cols_v, iv,
                        qq * QG, px, sprev)

    iul = idx_u[pl.ds((NT - 1) * LANES, LANES)]
    ivl = idx_v[pl.ds((NT - 1) * LANES, LANES)]
    for q in range(SPW - LAG, SPW):
        p = q & (SETS - 1)
        sbase = (NT - 1) * LANES + q * QG
        extract(ut_hbm, blku, sem_u, cols_u, iul, q * QG, p, sbase)
        extract(vt_hbm, blkv, sem_v, cols_v, ivl, q * QG, p, sbase)

    @pl.loop(0, NT)
    def _(g):
        bb = g * LANES
        acc = jnp.zeros((LANES,), jnp.float32)
        for d in range(RANK):
            uu = cols_u[pl.ds(d * BPW + bb, LANES)]
            vv = cols_v[pl.ds(d * BPW + bb, LANES)]
            acc = acc + uu * vv
        out_v[pl.ds(bb, LANES)] = acc

    pltpu.async_copy(out_v, out_hbm.at[pl.ds(base, BPW)], sem_o).wait()


def kernel(idxs, U, V):
    idxs = idxs.astype(jnp.int32)
    uidx = idxs[:, 0]
    vidx = idxs[:, 1]
    mesh = plsc.VectorSubcoreMesh(core_axis_name="c", subcore_axis_name="s")
    cp = pltpu.CompilerParams()
    if "needs_layout_passes" in pltpu.CompilerParams.__dataclass_fields__:
        cp = dataclasses.replace(cp, needs_layout_passes=False)
    if "use_tc_tiling_on_sc" in pltpu.CompilerParams.__dataclass_fields__:
        cp = dataclasses.replace(cp, use_tc_tiling_on_sc=True)
    run = pl.kernel(
        _dot_gather_body,
        out_type=jax.ShapeDtypeStruct((BATCH,), jnp.float32),
        mesh=mesh,
        scratch_types=[
            pltpu.VMEM((BPW,), jnp.int32),
            pltpu.VMEM((BPW,), jnp.int32),
            pltpu.VMEM((RANK * BPW,), jnp.float32),
            pltpu.VMEM((RANK * BPW,), jnp.float32),
            pltpu.VMEM((SETS, QG, RANK, BLK), jnp.float32),
            pltpu.VMEM((SETS, QG, RANK, BLK), jnp.float32),
            pltpu.VMEM((BPW,), jnp.float32),
            pltpu.SemaphoreType.DMA((SETS,)),
            pltpu.SemaphoreType.DMA((SETS,)),
            pltpu.SemaphoreType.DMA,
        ],
        compiler_params=cp,
    )
    return run(uidx, vidx, U.T, V.T)


# speedup vs baseline: 1.0754x; 1.0754x over previous
import dataclasses

import jax
import jax.numpy as jnp
from jax import lax
from jax.experimental import pallas as pl
from jax.experimental.pallas import tpu as pltpu
from jax.experimental.pallas import tpu_sc as plsc

BATCH = 16384
RANK = 32
NC = 2
NS = 16
LANES = 16
NW = NC * NS
BPW = BATCH // NW
QG = 2
SETS = 4
LAG = 3
SPW = LANES // QG
NT = BPW // LANES
BLK = 128


def _dot_gather_body(uidx_hbm, vidx_hbm, ut_hbm, vt_hbm, out_hbm,
                     idx_u, idx_v, cols_u, cols_v, blku, blkv, out_v,
                     sem_u, sem_v, sem_o):
    wid = lax.axis_index("s") * NC + lax.axis_index("c")
    base = wid * BPW

    cpu = pltpu.async_copy(uidx_hbm.at[pl.ds(base, BPW)], idx_u, sem_o)
    cpv = pltpu.async_copy(vidx_hbm.at[pl.ds(base, BPW)], idx_v, sem_o)
    cpu.wait()
    cpv.wait()

    iota = lax.iota(jnp.int32, LANES)

    def fire(t_hbm, blk, sem, i16, lane_base, p):
        for k in range(QG):
            c = pl.multiple_of((i16[lane_base + k] >> 7) * BLK, BLK)
            pltpu.async_copy(
                t_hbm.at[:, pl.ds(c, BLK)], blk.at[p, k], sem.at[p])

    def extract(t_hbm, blk, sem, cols, i16, lane_base, p, sbase):
        for k in range(QG):
            pltpu.make_async_copy(
                t_hbm.at[:, pl.ds(0, BLK)], blk.at[p, k], sem.at[p]).wait()
        pfull = jnp.full((LANES,), p, jnp.int32)
        for k in range(QG):
            b = sbase + k
            lane = jnp.full((LANES,), i16[lane_base + k] & (BLK - 1),
                            jnp.int32)
            kfull = jnp.full((LANES,), k, jnp.int32)
            lo = plsc.load_gather(blk, [pfull, kfull, iota, lane])
            hi = plsc.load_gather(blk, [pfull, kfull, LANES + iota, lane])
            plsc.store_scatter(cols, [iota * BPW + b], lo)
            plsc.store_scatter(cols, [(LANES + iota) * BPW + b], hi)

    @pl.loop(0, NT)
    def _(t):
        iu = idx_u[pl.ds(t * LANES, LANES)]
        iv = idx_v[pl.ds(t * LANES, LANES)]
        for q in range(SPW):
            p = q & (SETS - 1)
            fire(ut_hbm, blku, sem_u, iu, q * QG, p)
            fire(vt_hbm, blkv, sem_v, iv, q * QG, p)
            qq = q - LAG
            px = qq & (SETS - 1)
            sprev = t * LANES + qq * QG
            if qq < 0:
                @pl.when(t > 0)
                def _():
                    iup = idx_u[pl.ds((t - 1) * LANES, LANES)]
                    ivp = idx_v[pl.ds((t - 1) * LANES, LANES)]
                    extract(ut_hbm, blku, sem_u, cols_u, iup,
                            (qq + SPW) * QG, px, sprev)
                    extract(vt_hbm, blkv, sem_v, cols_v, ivp,
                            (qq + SPW) * QG, px, sprev)
            else:
                extract(ut_hbm, blku, sem_u, cols_u, iu,
                        qq * QG, px, sprev)
                extract(vt_hbm, blkv, sem_v, cols_v, iv,
                        qq * QG, px, sprev)

    iul = idx_u[pl.ds((NT - 1) * LANES, LANES)]
    ivl = idx_v[pl.ds((NT - 1) * LANES, LANES)]
    for q in range(SPW - LAG, SPW):
        p = q & (SETS - 1)
        sbase = (NT - 1) * LANES + q * QG
        extract(ut_hbm, blku, sem_u, cols_u, iul, q * QG, p, sbase)
        extract(vt_hbm, blkv, sem_v, cols_v, ivl, q * QG, p, sbase)

    @pl.loop(0, NT)
    def _(g):
        bb = g * LANES
        acc = jnp.zeros((LANES,), jnp.float32)
        for d in range(RANK):
            uu = cols_u[pl.ds(d * BPW + bb, LANES)]
            vv = cols_v[pl.ds(d * BPW + bb, LANES)]
            acc = acc + uu * vv
        out_v[pl.ds(bb, LANES)] = acc

    pltpu.async_copy(out_v, out_hbm.at[pl.ds(base, BPW)], sem_o).wait()


def kernel(idxs, U, V):
    idxs = idxs.astype(jnp.int32)
    uidx = idxs[:, 0]
    vidx = idxs[:, 1]
    mesh = plsc.VectorSubcoreMesh(core_axis_name="c", subcore_axis_name="s")
    cp = pltpu.CompilerParams()
    if "needs_layout_passes" in pltpu.CompilerParams.__dataclass_fields__:
        cp = dataclasses.replace(cp, needs_layout_passes=False)
    if "use_tc_tiling_on_sc" in pltpu.CompilerParams.__dataclass_fields__:
        cp = dataclasses.replace(cp, use_tc_tiling_on_sc=True)
    run = pl.kernel(
        _dot_gather_body,
        out_type=jax.ShapeDtypeStruct((BATCH,), jnp.float32),
        mesh=mesh,
        scratch_types=[
            pltpu.VMEM((BPW,), jnp.int32),
            pltpu.VMEM((BPW,), jnp.int32),
            pltpu.VMEM((RANK * BPW,), jnp.float32),
            pltpu.VMEM((RANK * BPW,), jnp.float32),
            pltpu.VMEM((SETS, QG, RANK, BLK), jnp.float32),
            pltpu.VMEM((SETS, QG, RANK, BLK), jnp.float32),
            pltpu.VMEM((BPW,), jnp.float32),
            pltpu.SemaphoreType.DMA((SETS,)),
            pltpu.SemaphoreType.DMA((SETS,)),
            pltpu.SemaphoreType.DMA,
        ],
        compiler_params=cp,
    )
    return run(uidx, vidx, U.T, V.T)
